# baseline (speedup 1.0000x reference)
import functools

import numpy as np
import jax
import jax.numpy as jnp
from jax import lax
from jax.experimental import pallas as pl
from jax.experimental.pallas import tpu as pltpu
from jax.experimental.pallas import tpu_sc as plsc

NB = 256
N_PTS = 2097152


_R1 = 64
_GRID1 = N_PTS // (_R1 * 256)

_C = np.zeros((768, 256), np.float32)
for _k in range(256):
    _C[3 * _k, _k] = 1.0


def _idx_body(xyz_ref, c_ref, idx_ref):
    v = xyz_ref[...]
    q = jnp.clip((v * 256.0 + 128.0).astype(jnp.int32), 0, NB - 1)
    inside = (jnp.abs(v) < 0.5).astype(jnp.int32)
    q1 = pltpu.roll(q, 767, 1)
    q2 = pltpu.roll(q, 766, 1)
    in1 = pltpu.roll(inside, 767, 1)
    in2 = pltpu.roll(inside, 766, 1)
    flat = (q * NB + q1) * NB + q2
    ok = (inside & in1 & in2) == 1
    flat = jnp.where(ok, flat, -1).astype(jnp.float32)
    idx = jnp.dot(flat, c_ref[...], precision=jax.lax.Precision.HIGHEST)
    idx_ref[...] = idx.astype(jnp.int32)


_idx_kernel = pl.pallas_call(
    _idx_body,
    grid=(_GRID1,),
    in_specs=[
        pl.BlockSpec((_R1, 768), lambda i: (i, 0)),
        pl.BlockSpec((768, 256), lambda i: (0, 0)),
    ],
    out_specs=pl.BlockSpec((_R1, 256), lambda i: (i, 0)),
    out_shape=jax.ShapeDtypeStruct((N_PTS // 256, 256), jnp.int32),
)


NC, NS, L = 2, 16, 16
NW = NC * NS
PER_W = N_PTS // NW
CHUNK = 2048
N_CHUNKS = PER_W // CHUNK
GATHER_SEG = 128
N_SEG = CHUNK // GATHER_SEG

_mesh = plsc.VectorSubcoreMesh(
    core_axis_name="c", subcore_axis_name="s", num_cores=NC, num_subcores=NS
)


@functools.partial(
    pl.kernel,
    out_type=jax.ShapeDtypeStruct((N_PTS, 4), jnp.float32),
    mesh=_mesh,
    scratch_types=[
        pltpu.VMEM((CHUNK,), jnp.int32),
        pltpu.VMEM((CHUNK, 4), jnp.float32),
        pltpu.SemaphoreType.DMA,
    ],
    compiler_params=pltpu.CompilerParams(use_tc_tiling_on_sc=False),
)
def _gather_sc(idx_hbm, vox_hbm, out_hbm, raw_v, rows_v, sem):
    wid = lax.axis_index("s") * NC + lax.axis_index("c")

    def chunk_body(ci, carry):
        base = wid * PER_W + ci * CHUNK
        pltpu.sync_copy(idx_hbm.at[pl.ds(base, CHUNK)], raw_v)

        copies = []
        for j in range(N_SEG):
            idx_slice = plsc.Indices(
                raw_v.at[pl.ds(j * GATHER_SEG, GATHER_SEG)], ignored_value=-1)
            copies.append(pltpu.async_copy(
                vox_hbm.at[idx_slice],
                rows_v.at[pl.ds(j * GATHER_SEG, GATHER_SEG)],
                sem,
            ))
        for cp in copies:
            cp.wait()

        pltpu.sync_copy(rows_v, out_hbm.at[pl.ds(base, CHUNK)])
        return carry

    lax.fori_loop(0, N_CHUNKS, chunk_body, 0)


_R2 = 64
_GRID2 = N_PTS // (_R2 * 256)

_E = np.zeros((256, 1024), np.float32)
for _p in range(256):
    for _c in range(4):
        _E[_p, 4 * _p + _c] = 1.0

_S3 = np.zeros((1024, 768), np.float32)
for _p in range(256):
    for _c in range(3):
        _S3[4 * _p + _c, 3 * _p + _c] = 1.0

_S1 = np.zeros((1024, 256), np.float32)
for _p in range(256):
    _S1[4 * _p + 3, _p] = 1.0


def _act_body(idx_ref, g_ref, e_ref, s3_ref, s1_ref, rgb_ref, sig_ref):
    condf = (idx_ref[...] >= 0).astype(jnp.float32)
    cond4 = jnp.dot(condf, e_ref[...])
    g = g_ref[...]
    m = jnp.where(cond4 > 0.5, g, 0.0)
    sg = jax.nn.sigmoid(m)
    rl = jnp.maximum(m, 0.0)
    lanes = jax.lax.broadcasted_iota(jnp.int32, (_R2, 1024), 1)
    act = jnp.where(lanes % 4 == 3, rl, sg)
    rgb_ref[...] = jnp.dot(act, s3_ref[...])
    sig_ref[...] = jnp.dot(act, s1_ref[...])


_act_kernel = pl.pallas_call(
    _act_body,
    grid=(_GRID2,),
    in_specs=[
        pl.BlockSpec((_R2, 256), lambda i: (i, 0)),
        pl.BlockSpec((_R2, 1024), lambda i: (i, 0)),
        pl.BlockSpec((256, 1024), lambda i: (0, 0)),
        pl.BlockSpec((1024, 768), lambda i: (0, 0)),
        pl.BlockSpec((1024, 256), lambda i: (0, 0)),
    ],
    out_specs=[
        pl.BlockSpec((_R2, 768), lambda i: (i, 0)),
        pl.BlockSpec((_R2, 256), lambda i: (i, 0)),
    ],
    out_shape=[
        jax.ShapeDtypeStruct((N_PTS // 256, 768), jnp.float32),
        jax.ShapeDtypeStruct((N_PTS // 256, 256), jnp.float32),
    ],
)


def kernel(xyz, voxels):
    xyz_blk = xyz.reshape(N_PTS // 256, 768)
    vox_flat = voxels.reshape(NB * NB * NB, 4)
    idx = _idx_kernel(xyz_blk, jnp.asarray(_C))
    gathered = _gather_sc(idx.reshape(N_PTS), vox_flat)
    rgb, sig = _act_kernel(
        idx, gathered.reshape(N_PTS // 256, 1024),
        jnp.asarray(_E), jnp.asarray(_S3), jnp.asarray(_S1))
    return rgb.reshape(N_PTS, 3), sig.reshape(N_PTS, 1)


# speedup vs baseline: 60.3847x; 60.3847x over previous
import functools

import jax
import jax.numpy as jnp
from jax import lax
from jax.experimental import pallas as pl
from jax.experimental.pallas import tpu as pltpu
from jax.experimental.pallas import tpu_sc as plsc

NB = 256
N_PTS = 2097152
MROWS = N_PTS // 128


_R1 = 256
_GRID1 = MROWS // _R1


def _idx_body(x_ref, y_ref, z_ref, idx_ref):
    x = x_ref[...]
    y = y_ref[...]
    z = z_ref[...]
    cond = ((jnp.abs(x) < 0.5) & (jnp.abs(y) < 0.5) & (jnp.abs(z) < 0.5))
    ix = jnp.clip((x * 256.0 + 128.0).astype(jnp.int32), 0, NB - 1)
    iy = jnp.clip((y * 256.0 + 128.0).astype(jnp.int32), 0, NB - 1)
    iz = jnp.clip((z * 256.0 + 128.0).astype(jnp.int32), 0, NB - 1)
    base = (ix * NB + iy) * 1024 + (iz >> 7) * 512 + (iz & 127)
    idx_ref[...] = jnp.where(cond, base, -1)


_idx_kernel = pl.pallas_call(
    _idx_body,
    grid=(_GRID1,),
    in_specs=[pl.BlockSpec((_R1, 128), lambda i: (i, 0))] * 3,
    out_specs=pl.BlockSpec((_R1, 128), lambda i: (i, 0)),
    out_shape=jax.ShapeDtypeStruct((MROWS, 128), jnp.int32),
)


NC, NS, L = 2, 16, 16
NW = NC * NS
PER_W = N_PTS // NW
CHUNK = 2048
N_CHUNKS = PER_W // CHUNK
GROUPS = CHUNK // L
GATHER_SEG = 128
N_SEG = CHUNK // GATHER_SEG

_mesh = plsc.VectorSubcoreMesh(
    core_axis_name="c", subcore_axis_name="s", num_cores=NC, num_subcores=NS
)


@functools.partial(
    pl.kernel,
    out_type=jax.ShapeDtypeStruct((4, N_PTS), jnp.float32),
    mesh=_mesh,
    scratch_types=[
        pltpu.VMEM((CHUNK,), jnp.int32),
        pltpu.VMEM((CHUNK,), jnp.int32),
        pltpu.VMEM((CHUNK,), jnp.int32),
        pltpu.VMEM((CHUNK,), jnp.int32),
        pltpu.VMEM((CHUNK,), jnp.float32),
        pltpu.VMEM((CHUNK,), jnp.float32),
        pltpu.VMEM((CHUNK,), jnp.float32),
        pltpu.VMEM((CHUNK,), jnp.float32),
        pltpu.SemaphoreType.DMA,
    ],
    compiler_params=pltpu.CompilerParams(use_tc_tiling_on_sc=False),
)
def _gather_sc(idx_hbm, vox_hbm, out_hbm,
               ib_v, o1_v, o2_v, o3_v, c0_v, c1_v, c2_v, c3_v, sem):
    wid = lax.axis_index("s") * NC + lax.axis_index("c")

    def chunk_body(ci, carry):
        base = wid * PER_W + ci * CHUNK
        pltpu.sync_copy(idx_hbm.at[pl.ds(base, CHUNK)], ib_v)

        copies = []

        def fire(ob, dst, ig):
            for j in range(N_SEG):
                sl = pl.ds(j * GATHER_SEG, GATHER_SEG)
                copies.append(pltpu.async_copy(
                    vox_hbm.at[plsc.Indices(ob.at[sl], ignored_value=ig)],
                    dst.at[sl], sem))

        fire(ib_v, c0_v, -1)

        def off_body(g, carry2, ob=None, c=0):
            w = ib_v[pl.ds(g * L, L)]
            ob[pl.ds(g * L, L)] = w + 128 * c
            return carry2

        for c, (ob, dst) in enumerate(
                [(o1_v, c1_v), (o2_v, c2_v), (o3_v, c3_v)], start=1):
            lax.fori_loop(0, GROUPS,
                          functools.partial(off_body, ob=ob, c=c), 0,
                          unroll=4)
            fire(ob, dst, 128 * c - 1)

        for cp in copies:
            cp.wait()

        for c, src in enumerate([c0_v, c1_v, c2_v, c3_v]):
            pltpu.sync_copy(src, out_hbm.at[c, pl.ds(base, CHUNK)])
        return carry

    lax.fori_loop(0, N_CHUNKS, chunk_body, 0)


_R2 = 256
_GRID2 = MROWS // _R2


def _act_body(idx_ref, r_ref, g_ref, b_ref, s_ref,
              ro_ref, go_ref, bo_ref, so_ref):
    cond = idx_ref[...] >= 0
    zero = jnp.float32(0.0)
    r = jnp.where(cond, r_ref[...], zero)
    g = jnp.where(cond, g_ref[...], zero)
    b = jnp.where(cond, b_ref[...], zero)
    s = jnp.where(cond, s_ref[...], zero)
    ro_ref[...] = jax.nn.sigmoid(r)
    go_ref[...] = jax.nn.sigmoid(g)
    bo_ref[...] = jax.nn.sigmoid(b)
    so_ref[...] = jnp.maximum(s, zero)


_act_kernel = pl.pallas_call(
    _act_body,
    grid=(_GRID2,),
    in_specs=[
        pl.BlockSpec((_R2, 128), lambda i: (i, 0)),
        pl.BlockSpec((_R2, 128), lambda i: (i, 0)),
        pl.BlockSpec((_R2, 128), lambda i: (i + _GRID2, 0)),
        pl.BlockSpec((_R2, 128), lambda i: (i + 2 * _GRID2, 0)),
        pl.BlockSpec((_R2, 128), lambda i: (i + 3 * _GRID2, 0)),
    ],
    out_specs=[pl.BlockSpec((_R2, 128), lambda i: (i, 0))] * 4,
    out_shape=[jax.ShapeDtypeStruct((MROWS, 128), jnp.float32)] * 4,
)


def kernel(xyz, voxels):
    x = xyz[:, 0].reshape(MROWS, 128)
    y = xyz[:, 1].reshape(MROWS, 128)
    z = xyz[:, 2].reshape(MROWS, 128)
    vox_lin = (voxels.reshape(NB, NB, 2, 128, 4)
               .transpose(0, 1, 2, 4, 3)
               .reshape(NB * NB * NB * 4))
    idx = _idx_kernel(x, y, z)
    g4 = _gather_sc(idx.reshape(N_PTS), vox_lin)
    g4v = g4.reshape(4 * MROWS, 128)
    rp, gp, bp, sp = _act_kernel(idx, g4v, g4v, g4v, g4v)
    rgb = jnp.stack(
        [rp.reshape(N_PTS), gp.reshape(N_PTS), bp.reshape(N_PTS)], axis=1)
    return rgb, sp.reshape(N_PTS, 1)


# speedup vs baseline: 85.5157x; 1.4162x over previous
import functools

import jax
import jax.numpy as jnp
from jax import lax
from jax.experimental import pallas as pl
from jax.experimental.pallas import tpu as pltpu
from jax.experimental.pallas import tpu_sc as plsc

NB = 256
N_PTS = 2097152
MROWS = N_PTS // 128


_R1 = 256
_GRID1 = MROWS // _R1


def _idx_body(x_ref, y_ref, z_ref, idx_ref):
    x = x_ref[...]
    y = y_ref[...]
    z = z_ref[...]
    cond = ((jnp.abs(x) < 0.5) & (jnp.abs(y) < 0.5) & (jnp.abs(z) < 0.5))
    ix = jnp.clip((x * 256.0 + 128.0).astype(jnp.int32), 0, NB - 1)
    iy = jnp.clip((y * 256.0 + 128.0).astype(jnp.int32), 0, NB - 1)
    iz = jnp.clip((z * 256.0 + 128.0).astype(jnp.int32), 0, NB - 1)
    base = (ix * NB + iy) * 1024 + (iz >> 7) * 512 + (iz & 127)
    idx_ref[...] = jnp.where(cond, base, -1)


_idx_kernel = pl.pallas_call(
    _idx_body,
    grid=(_GRID1,),
    in_specs=[pl.BlockSpec((_R1, 128), lambda i: (i, 0))] * 3,
    out_specs=pl.BlockSpec((_R1, 128), lambda i: (i, 0)),
    out_shape=jax.ShapeDtypeStruct((MROWS, 128), jnp.int32),
)


NC, NS, L = 2, 16, 16
NW = NC * NS
PER_W = N_PTS // NW
CHUNK = 2048
N_CHUNKS = PER_W // CHUNK
N_PAIRS = N_CHUNKS // 2
GROUPS = CHUNK // L
GATHER_SEG = 512
N_SEG = CHUNK // GATHER_SEG

_mesh = plsc.VectorSubcoreMesh(
    core_axis_name="c", subcore_axis_name="s", num_cores=NC, num_subcores=NS
)


@functools.partial(
    pl.kernel,
    out_type=jax.ShapeDtypeStruct((4, N_PTS), jnp.float32),
    mesh=_mesh,
    scratch_types=[
        pltpu.VMEM((2, CHUNK), jnp.int32),
        pltpu.VMEM((2, CHUNK), jnp.int32),
        pltpu.VMEM((2, CHUNK), jnp.int32),
        pltpu.VMEM((2, CHUNK), jnp.int32),
        pltpu.VMEM((2, CHUNK), jnp.float32),
        pltpu.VMEM((2, CHUNK), jnp.float32),
        pltpu.VMEM((2, CHUNK), jnp.float32),
        pltpu.VMEM((2, CHUNK), jnp.float32),
        pltpu.SemaphoreType.DMA,
        pltpu.SemaphoreType.DMA,
        pltpu.SemaphoreType.DMA,
        pltpu.SemaphoreType.DMA,
        pltpu.SemaphoreType.DMA,
        pltpu.SemaphoreType.DMA,
    ],
    compiler_params=pltpu.CompilerParams(use_tc_tiling_on_sc=False),
)
def _gather_sc(idx_hbm, vox_hbm, out_hbm,
               ib_v, o1_v, o2_v, o3_v, c0_v, c1_v, c2_v, c3_v,
               isem0, isem1, gsem0, gsem1, osem0, osem1):
    wid = lax.axis_index("s") * NC + lax.axis_index("c")
    isem = (isem0, isem1)
    gsem = (gsem0, gsem1)
    osem = (osem0, osem1)

    def cbase(ci):
        return wid * PER_W + ci * CHUNK

    def gather_copies(par, ci):
        out = []
        for c, ob in enumerate([ib_v, o1_v, o2_v, o3_v]):
            dst = (c0_v, c1_v, c2_v, c3_v)[c]
            for j in range(N_SEG):
                sl = pl.ds(j * GATHER_SEG, GATHER_SEG)
                out.append(pltpu.make_async_copy(
                    vox_hbm.at[plsc.Indices(ob.at[par].at[sl],
                                            ignored_value=128 * c - 1)],
                    dst.at[par].at[sl], gsem[par]))
        return out

    def out_copies(par, ci):
        return [pltpu.make_async_copy(
                    (c0_v, c1_v, c2_v, c3_v)[c].at[par],
                    out_hbm.at[c, pl.ds(cbase(ci), CHUNK)], osem[par])
                for c in range(4)]

    def in_copy(par, ci):
        return pltpu.make_async_copy(
            idx_hbm.at[pl.ds(cbase(ci), CHUNK)], ib_v.at[par], isem[par])

    in_copy(0, 0).start()

    def pair_body(p, carry):
        for par in (0, 1):
            ci = 2 * p + par
            oth = 1 - par
            in_copy(par, ci).wait()
            def off_body(g, carry2, ob=None, c=0):
                w = ib_v[par, pl.ds(g * L, L)]
                ob[par, pl.ds(g * L, L)] = w + 128 * c
                return carry2
            for c, ob in enumerate([o1_v, o2_v, o3_v], start=1):
                lax.fori_loop(0, GROUPS,
                              functools.partial(off_body, ob=ob, c=c), 0,
                              unroll=4)
            @pl.when(ci >= 2)
            def _():
                for cp in out_copies(par, ci - 2):
                    cp.wait()
            for cp in gather_copies(par, ci):
                cp.start()
            @pl.when(ci >= 1)
            def _():
                for cp in gather_copies(oth, ci - 1):
                    cp.wait()
                for cp in out_copies(oth, ci - 1):
                    cp.start()
            @pl.when(ci + 1 <= N_CHUNKS - 1)
            def _():
                in_copy(oth, ci + 1).start()
        return carry

    lax.fori_loop(0, N_PAIRS, pair_body, 0)

    last = N_CHUNKS - 1
    for cp in gather_copies(1, last):
        cp.wait()
    for cp in out_copies(1, last):
        cp.start()
    for cp in out_copies(0, last - 1):
        cp.wait()
    for cp in out_copies(1, last):
        cp.wait()


_R2 = 256
_GRID2 = MROWS // _R2


def _act_body(idx_ref, r_ref, g_ref, b_ref, s_ref,
              ro_ref, go_ref, bo_ref, so_ref):
    cond = idx_ref[...] >= 0
    zero = jnp.float32(0.0)
    r = jnp.where(cond, r_ref[...], zero)
    g = jnp.where(cond, g_ref[...], zero)
    b = jnp.where(cond, b_ref[...], zero)
    s = jnp.where(cond, s_ref[...], zero)
    ro_ref[...] = jax.nn.sigmoid(r)
    go_ref[...] = jax.nn.sigmoid(g)
    bo_ref[...] = jax.nn.sigmoid(b)
    so_ref[...] = jnp.maximum(s, zero)


_act_kernel = pl.pallas_call(
    _act_body,
    grid=(_GRID2,),
    in_specs=[
        pl.BlockSpec((_R2, 128), lambda i: (i, 0)),
        pl.BlockSpec((_R2, 128), lambda i: (i, 0)),
        pl.BlockSpec((_R2, 128), lambda i: (i + _GRID2, 0)),
        pl.BlockSpec((_R2, 128), lambda i: (i + 2 * _GRID2, 0)),
        pl.BlockSpec((_R2, 128), lambda i: (i + 3 * _GRID2, 0)),
    ],
    out_specs=[pl.BlockSpec((_R2, 128), lambda i: (i, 0))] * 4,
    out_shape=[jax.ShapeDtypeStruct((MROWS, 128), jnp.float32)] * 4,
)


def kernel(xyz, voxels):
    x = xyz[:, 0].reshape(MROWS, 128)
    y = xyz[:, 1].reshape(MROWS, 128)
    z = xyz[:, 2].reshape(MROWS, 128)
    vox_lin = (voxels.reshape(NB, NB, 2, 128, 4)
               .transpose(0, 1, 2, 4, 3)
               .reshape(NB * NB * NB * 4))
    idx = _idx_kernel(x, y, z)
    g4 = _gather_sc(idx.reshape(N_PTS), vox_lin)
    g4v = g4.reshape(4 * MROWS, 128)
    rp, gp, bp, sp = _act_kernel(idx, g4v, g4v, g4v, g4v)
    rgb = jnp.stack(
        [rp.reshape(N_PTS), gp.reshape(N_PTS), bp.reshape(N_PTS)], axis=1)
    return rgb, sp.reshape(N_PTS, 1)


# speedup vs baseline: 85.6314x; 1.0014x over previous
import functools

import jax
import jax.numpy as jnp
from jax import lax
from jax.experimental import pallas as pl
from jax.experimental.pallas import tpu as pltpu
from jax.experimental.pallas import tpu_sc as plsc

NB = 256
N_PTS = 2097152
MROWS = N_PTS // 128


_R1 = 256
_GRID1 = MROWS // _R1


def _idx_body(x_ref, y_ref, z_ref, idx_ref):
    x = x_ref[...]
    y = y_ref[...]
    z = z_ref[...]
    cond = ((jnp.abs(x) < 0.5) & (jnp.abs(y) < 0.5) & (jnp.abs(z) < 0.5))
    ix = jnp.clip((x * 256.0 + 128.0).astype(jnp.int32), 0, NB - 1)
    iy = jnp.clip((y * 256.0 + 128.0).astype(jnp.int32), 0, NB - 1)
    iz = jnp.clip((z * 256.0 + 128.0).astype(jnp.int32), 0, NB - 1)
    base = (ix * NB + iy) * 1024 + (iz >> 7) * 512 + (iz & 127)
    idx_ref[...] = jnp.where(cond, base, -1)


_idx_kernel = pl.pallas_call(
    _idx_body,
    grid=(_GRID1,),
    in_specs=[pl.BlockSpec((_R1, 128), lambda i: (i, 0))] * 3,
    out_specs=pl.BlockSpec((_R1, 128), lambda i: (i, 0)),
    out_shape=jax.ShapeDtypeStruct((MROWS, 128), jnp.int32),
)


NC, NS, L = 2, 16, 16
NW = NC * NS
PER_W = N_PTS // NW
CHUNK = 2048
N_CHUNKS = PER_W // CHUNK
N_PAIRS = N_CHUNKS // 2
GROUPS = CHUNK // L
GATHER_SEG = 2048
N_SEG = CHUNK // GATHER_SEG

_mesh = plsc.VectorSubcoreMesh(
    core_axis_name="c", subcore_axis_name="s", num_cores=NC, num_subcores=NS
)


@functools.partial(
    pl.kernel,
    out_type=jax.ShapeDtypeStruct((4, N_PTS), jnp.float32),
    mesh=_mesh,
    scratch_types=[
        pltpu.VMEM((2, CHUNK), jnp.int32),
        pltpu.VMEM((2, CHUNK), jnp.int32),
        pltpu.VMEM((2, CHUNK), jnp.int32),
        pltpu.VMEM((2, CHUNK), jnp.int32),
        pltpu.VMEM((2, CHUNK), jnp.float32),
        pltpu.VMEM((2, CHUNK), jnp.float32),
        pltpu.VMEM((2, CHUNK), jnp.float32),
        pltpu.VMEM((2, CHUNK), jnp.float32),
        pltpu.SemaphoreType.DMA,
        pltpu.SemaphoreType.DMA,
        pltpu.SemaphoreType.DMA,
        pltpu.SemaphoreType.DMA,
        pltpu.SemaphoreType.DMA,
        pltpu.SemaphoreType.DMA,
    ],
    compiler_params=pltpu.CompilerParams(use_tc_tiling_on_sc=False),
)
def _gather_sc(idx_hbm, vox_hbm, out_hbm,
               ib_v, o1_v, o2_v, o3_v, c0_v, c1_v, c2_v, c3_v,
               isem0, isem1, gsem0, gsem1, osem0, osem1):
    wid = lax.axis_index("s") * NC + lax.axis_index("c")
    isem = (isem0, isem1)
    gsem = (gsem0, gsem1)
    osem = (osem0, osem1)

    def cbase(ci):
        return wid * PER_W + ci * CHUNK

    def gather_copies(par, ci):
        out = []
        for c, ob in enumerate([ib_v, o1_v, o2_v, o3_v]):
            dst = (c0_v, c1_v, c2_v, c3_v)[c]
            for j in range(N_SEG):
                sl = pl.ds(j * GATHER_SEG, GATHER_SEG)
                out.append(pltpu.make_async_copy(
                    vox_hbm.at[plsc.Indices(ob.at[par].at[sl],
                                            ignored_value=128 * c - 1)],
                    dst.at[par].at[sl], gsem[par]))
        return out

    def out_copies(par, ci):
        return [pltpu.make_async_copy(
                    (c0_v, c1_v, c2_v, c3_v)[c].at[par],
                    out_hbm.at[c, pl.ds(cbase(ci), CHUNK)], osem[par])
                for c in range(4)]

    def in_copy(par, ci):
        return pltpu.make_async_copy(
            idx_hbm.at[pl.ds(cbase(ci), CHUNK)], ib_v.at[par], isem[par])

    in_copy(0, 0).start()

    def pair_body(p, carry):
        for par in (0, 1):
            ci = 2 * p + par
            oth = 1 - par
            in_copy(par, ci).wait()
            def off_body(g, carry2, ob=None, c=0):
                w = ib_v[par, pl.ds(g * L, L)]
                ob[par, pl.ds(g * L, L)] = w + 128 * c
                return carry2
            for c, ob in enumerate([o1_v, o2_v, o3_v], start=1):
                lax.fori_loop(0, GROUPS,
                              functools.partial(off_body, ob=ob, c=c), 0,
                              unroll=4)
            @pl.when(ci >= 2)
            def _():
                for cp in out_copies(par, ci - 2):
                    cp.wait()
            for cp in gather_copies(par, ci):
                cp.start()
            @pl.when(ci >= 1)
            def _():
                for cp in gather_copies(oth, ci - 1):
                    cp.wait()
                for cp in out_copies(oth, ci - 1):
                    cp.start()
            @pl.when(ci + 1 <= N_CHUNKS - 1)
            def _():
                in_copy(oth, ci + 1).start()
        return carry

    lax.fori_loop(0, N_PAIRS, pair_body, 0)

    last = N_CHUNKS - 1
    for cp in gather_copies(1, last):
        cp.wait()
    for cp in out_copies(1, last):
        cp.start()
    for cp in out_copies(0, last - 1):
        cp.wait()
    for cp in out_copies(1, last):
        cp.wait()


_R2 = 256
_GRID2 = MROWS // _R2


def _act_body(idx_ref, r_ref, g_ref, b_ref, s_ref,
              ro_ref, go_ref, bo_ref, so_ref):
    cond = idx_ref[...] >= 0
    zero = jnp.float32(0.0)
    r = jnp.where(cond, r_ref[...], zero)
    g = jnp.where(cond, g_ref[...], zero)
    b = jnp.where(cond, b_ref[...], zero)
    s = jnp.where(cond, s_ref[...], zero)
    ro_ref[...] = jax.nn.sigmoid(r)
    go_ref[...] = jax.nn.sigmoid(g)
    bo_ref[...] = jax.nn.sigmoid(b)
    so_ref[...] = jnp.maximum(s, zero)


_act_kernel = pl.pallas_call(
    _act_body,
    grid=(_GRID2,),
    in_specs=[
        pl.BlockSpec((_R2, 128), lambda i: (i, 0)),
        pl.BlockSpec((_R2, 128), lambda i: (i, 0)),
        pl.BlockSpec((_R2, 128), lambda i: (i + _GRID2, 0)),
        pl.BlockSpec((_R2, 128), lambda i: (i + 2 * _GRID2, 0)),
        pl.BlockSpec((_R2, 128), lambda i: (i + 3 * _GRID2, 0)),
    ],
    out_specs=[pl.BlockSpec((_R2, 128), lambda i: (i, 0))] * 4,
    out_shape=[jax.ShapeDtypeStruct((MROWS, 128), jnp.float32)] * 4,
)


def kernel(xyz, voxels):
    x = xyz[:, 0].reshape(MROWS, 128)
    y = xyz[:, 1].reshape(MROWS, 128)
    z = xyz[:, 2].reshape(MROWS, 128)
    vox_lin = (voxels.reshape(NB, NB, 2, 128, 4)
               .transpose(0, 1, 2, 4, 3)
               .reshape(NB * NB * NB * 4))
    idx = _idx_kernel(x, y, z)
    g4 = _gather_sc(idx.reshape(N_PTS), vox_lin)
    g4v = g4.reshape(4 * MROWS, 128)
    rp, gp, bp, sp = _act_kernel(idx, g4v, g4v, g4v, g4v)
    rgb = jnp.stack(
        [rp.reshape(N_PTS), gp.reshape(N_PTS), bp.reshape(N_PTS)], axis=1)
    return rgb, sp.reshape(N_PTS, 1)
